# baseline (speedup 1.0000x reference)
import functools

import jax
import jax.numpy as jnp
from jax import lax
from jax.experimental import pallas as pl
from jax.experimental.pallas import tpu as pltpu
from jax.experimental.pallas import tpu_sc as plsc

N_TOK = 16384
K = 1024
D = 2
T = 2048
G = N_TOK // T

NW = 32
TPW = N_TOK // NW
L = 16
CH = TPW // L


def _dist_body(x_ref, w_ref, idx_ref, perp_ref, loss_ref, hist_ref, acc_ref):
    i = pl.program_id(0)
    x = x_ref[...]
    w = w_ref[...]

    x2 = jnp.sum(x * x, axis=1, keepdims=True)
    w2 = jnp.sum(w * w, axis=1)
    m2 = lax.dot_general(x, w * (-2.0), (((1,), (1,)), ((), ())),
                         preferred_element_type=jnp.float32)
    d = (x2 + w2[None, :]) + m2

    mind = jnp.min(d, axis=1, keepdims=True)
    kio = lax.broadcasted_iota(jnp.int32, (T, K), 1)
    idx = jnp.min(jnp.where(d == mind, kio, K), axis=1)
    idx_ref[...] = idx.reshape(T // 128, 128)

    onehot = jnp.where(kio == idx[:, None], 1.0, 0.0)
    histp = jnp.sum(onehot, axis=0).reshape(K // 128, 128)

    part = jnp.sum(mind)

    @pl.when(i == 0)
    def _():
        acc_ref[0] = part
        hist_ref[...] = histp

    @pl.when(i > 0)
    def _():
        acc_ref[0] = acc_ref[0] + part
        hist_ref[...] = hist_ref[...] + histp

    @pl.when(i == G - 1)
    def _():
        loss_ref[0, 0] = acc_ref[0] * (1.0 / (N_TOK * D))
        avg = hist_ref[...] * (1.0 / N_TOK)
        ent = jnp.sum(avg * jnp.log(avg + 1e-10))
        perp_ref[0, 0] = jnp.exp(-ent)


def _tc_indices(inputs, weight):
    return pl.pallas_call(
        _dist_body,
        grid=(G,),
        in_specs=[
            pl.BlockSpec((T, D), lambda i: (i, 0)),
            pl.BlockSpec((K, D), lambda i: (0, 0)),
        ],
        out_specs=[
            pl.BlockSpec((T // 128, 128), lambda i: (i, 0)),
            pl.BlockSpec((1, 1), lambda i: (0, 0), memory_space=pltpu.SMEM),
            pl.BlockSpec((1, 1), lambda i: (0, 0), memory_space=pltpu.SMEM),
        ],
        out_shape=[
            jax.ShapeDtypeStruct((N_TOK // 128, 128), jnp.int32),
            jax.ShapeDtypeStruct((1, 1), jnp.float32),
            jax.ShapeDtypeStruct((1, 1), jnp.float32),
        ],
        scratch_shapes=[
            pltpu.VMEM((K // 128, 128), jnp.float32),
            pltpu.SMEM((1,), jnp.float32),
        ],
    )(inputs, weight)


def _sc_body(idx_hbm, w_hbm, st_hbm, idx_v, w_v, st_v):
    wid = lax.axis_index("s") * 2 + lax.axis_index("c")
    base = wid * TPW

    pltpu.sync_copy(idx_hbm.at[pl.ds(base, TPW)], idx_v)
    pltpu.sync_copy(w_hbm, w_v)

    lane = lax.broadcasted_iota(jnp.int32, (L,), 0)
    for c in range(CH):
        iv = idx_v[pl.ds(c * L, L)]
        w0 = iv * 2
        q0 = plsc.load_gather(w_v, [w0])
        q1 = plsc.load_gather(w_v, [w0 + 1])
        p0 = (lane + c * L) * 2
        plsc.store_scatter(st_v, [p0], q0)
        plsc.store_scatter(st_v, [p0 + 1], q1)

    pltpu.sync_copy(st_v, st_hbm.at[pl.ds(2 * base, 2 * TPW)])


_sc_quantize = functools.partial(
    pl.kernel,
    mesh=plsc.VectorSubcoreMesh(core_axis_name="c", subcore_axis_name="s"),
    out_type=jax.ShapeDtypeStruct((2 * N_TOK,), jnp.float32),
    scratch_types=[
        pltpu.VMEM((TPW,), jnp.int32),
        pltpu.VMEM((2 * K,), jnp.float32),
        pltpu.VMEM((2 * TPW,), jnp.float32),
    ],
    compiler_params=pltpu.CompilerParams(needs_layout_passes=False),
)(_sc_body)


def kernel(inputs, weight):
    idx, perp, loss = _tc_indices(inputs, weight)
    st_flat = _sc_quantize(idx.reshape(N_TOK), weight.reshape(2 * K))
    return st_flat.reshape(N_TOK, D), perp[0, 0], loss[0, 0]


# speedup vs baseline: 1.0634x; 1.0634x over previous
import functools

import jax
import jax.numpy as jnp
from jax import lax
from jax.experimental import pallas as pl
from jax.experimental.pallas import tpu as pltpu
from jax.experimental.pallas import tpu_sc as plsc

N_TOK = 16384
K = 1024
D = 2
T = 2048
G = N_TOK // T

NW = 32
TPW = N_TOK // NW
L = 16
CH = TPW // L


def _dist_body(x_ref, w_ref, idx_ref, loss_ref, acc_ref):
    i = pl.program_id(0)
    x = x_ref[...]
    w = w_ref[...]

    x2 = jnp.sum(x * x, axis=1, keepdims=True)
    w2 = jnp.sum(w * w, axis=1)
    m2 = lax.dot_general(x, w * (-2.0), (((1,), (1,)), ((), ())),
                         preferred_element_type=jnp.float32)
    d = (x2 + w2[None, :]) + m2

    mind = jnp.min(d, axis=1, keepdims=True)
    kio = lax.broadcasted_iota(jnp.int32, (T, K), 1)
    idx = jnp.min(jnp.where(d == mind, kio, K), axis=1)
    idx_ref[...] = idx.reshape(T // 128, 128)

    part = jnp.sum(mind)

    @pl.when(i == 0)
    def _():
        acc_ref[0] = part

    @pl.when(i > 0)
    def _():
        acc_ref[0] = acc_ref[0] + part

    @pl.when(i == G - 1)
    def _():
        loss_ref[0, 0] = acc_ref[0] * (1.0 / (N_TOK * D))


def _tc_indices(inputs, weight):
    return pl.pallas_call(
        _dist_body,
        grid=(G,),
        in_specs=[
            pl.BlockSpec((T, D), lambda i: (i, 0)),
            pl.BlockSpec((K, D), lambda i: (0, 0)),
        ],
        out_specs=[
            pl.BlockSpec((T // 128, 128), lambda i: (i, 0)),
            pl.BlockSpec((1, 1), lambda i: (0, 0), memory_space=pltpu.SMEM),
        ],
        out_shape=[
            jax.ShapeDtypeStruct((N_TOK // 128, 128), jnp.int32),
            jax.ShapeDtypeStruct((1, 1), jnp.float32),
        ],
        scratch_shapes=[
            pltpu.SMEM((1,), jnp.float32),
        ],
    )(inputs, weight)


def _sc_body(idx_hbm, w_hbm, st_hbm, hist_hbm,
             idx_v, w_v, st_v, hist_v):
    wid = lax.axis_index("s") * 2 + lax.axis_index("c")
    base = wid * TPW

    pltpu.sync_copy(idx_hbm.at[pl.ds(base, TPW)], idx_v)
    pltpu.sync_copy(w_hbm, w_v)

    zf = jnp.zeros((L,), jnp.float32)
    for c in range(K // L):
        hist_v[pl.ds(c * L, L)] = zf

    lane = lax.broadcasted_iota(jnp.int32, (L,), 0)
    onef = jnp.ones((L,), jnp.float32)
    for c in range(CH):
        iv = idx_v[pl.ds(c * L, L)]
        w0 = iv * 2
        q0 = plsc.load_gather(w_v, [w0])
        q1 = plsc.load_gather(w_v, [w0 + 1])
        p0 = (lane + c * L) * 2
        plsc.store_scatter(st_v, [p0], q0)
        plsc.store_scatter(st_v, [p0 + 1], q1)
        plsc.addupdate_scatter(hist_v, [iv], onef)

    pltpu.sync_copy(st_v, st_hbm.at[pl.ds(2 * base, 2 * TPW)])
    pltpu.sync_copy(hist_v, hist_hbm.at[wid])


_sc_quantize = functools.partial(
    pl.kernel,
    mesh=plsc.VectorSubcoreMesh(core_axis_name="c", subcore_axis_name="s"),
    out_type=[
        jax.ShapeDtypeStruct((2 * N_TOK,), jnp.float32),
        jax.ShapeDtypeStruct((NW, K), jnp.float32),
    ],
    scratch_types=[
        pltpu.VMEM((TPW,), jnp.int32),
        pltpu.VMEM((2 * K,), jnp.float32),
        pltpu.VMEM((2 * TPW,), jnp.float32),
        pltpu.VMEM((K,), jnp.float32),
    ],
    compiler_params=pltpu.CompilerParams(needs_layout_passes=False),
)(_sc_body)


def _fin_body(hist_ref, perp_out):
    avg = jnp.sum(hist_ref[...], axis=0, keepdims=True) * (1.0 / N_TOK)
    ent = jnp.sum(avg * jnp.log(avg + 1e-10))
    perp_out[...] = jnp.exp(-ent)[None, None]


def _finalize(hist):
    return pl.pallas_call(
        _fin_body,
        out_shape=jax.ShapeDtypeStruct((1, 1), jnp.float32),
    )(hist)


def kernel(inputs, weight):
    idx, loss = _tc_indices(inputs, weight)
    st_flat, hist = _sc_quantize(idx.reshape(N_TOK), weight.reshape(2 * K))
    perp = _finalize(hist)
    return st_flat.reshape(N_TOK, D), perp[0, 0], loss[0, 0]
